# baseline (speedup 1.0000x reference)
import functools

import jax
import jax.numpy as jnp
from jax import lax
from jax.experimental import pallas as pl
from jax.experimental.pallas import tpu as pltpu
from jax.experimental.pallas import tpu_sc as plsc

_H = 200
_BATCH = 4096
_D = 64
_BB = 128


@functools.lru_cache(maxsize=None)
def _make():
    info = plsc.get_sparse_core_info()
    nw = info.num_cores * info.num_subcores
    assert nw * _BB == _BATCH

    mesh = plsc.VectorSubcoreMesh(core_axis_name="c", subcore_axis_name="s")

    @functools.partial(
        pl.kernel,
        mesh=mesh,
        out_type=jax.ShapeDtypeStruct((_H, _D, _BATCH), jnp.float32),
        scratch_types=[
            pltpu.VMEM((_H, _BB), jnp.int32),
            pltpu.VMEM((_BB,), jnp.int32),
            pltpu.VMEM((_BB,), jnp.int32),
            pltpu.VMEM((_BB,), jnp.int32),
            pltpu.VMEM((_BB,), jnp.int32),
            pltpu.VMEM((_BB, 128), jnp.float32),
            pltpu.VMEM((_BB, 128), jnp.float32),
            pltpu.VMEM((_D, _BB), jnp.float32),
            pltpu.VMEM((_D, _BB), jnp.float32),
            pltpu.SemaphoreType.DMA,
            pltpu.SemaphoreType.DMA,
            pltpu.SemaphoreType.DMA,
            pltpu.SemaphoreType.DMA,
        ],
        compiler_params=pltpu.CompilerParams(
            use_tc_tiling_on_sc=True, needs_layout_passes=False),
    )
    def k(xst_hbm, table_hbm, out_hbm,
          idxt, i0, i1, p0, p1, g0, g1, o0, o1, sg0, sg1, sw0, sw1):
        wid = lax.axis_index("s") * info.num_cores + lax.axis_index("c")
        col = wid * _BB
        idx2 = (i0, i1)
        p64 = (p0, p1)
        grows = (g0, g1)
        ot = (o0, o1)
        sg = (sg0, sg1)
        sw = (sw0, sw1)

        iota = lax.iota(jnp.int32, 16)
        lanes = [iota + 16 * j for j in range(8)]

        pltpu.sync_copy(xst_hbm.at[:, pl.ds(col, _BB)], idxt)

        def prep(h, b):
            hv = jnp.full((16,), h, jnp.int32)
            for j in range(8):
                v = plsc.load_gather(idxt, [hv, lanes[j]])
                idx2[b][pl.ds(16 * j, 16)] = v >> 1
                p64[b][pl.ds(16 * j, 16)] = (v & 1) * 64

        def g_start(b):
            pltpu.async_copy(table_hbm.at[idx2[b]], grows[b], sg[b])

        def g_wait(b):
            pltpu.make_async_copy(table_hbm.at[idx2[b]], grows[b], sg[b]).wait()

        def w_start(h, b):
            pltpu.async_copy(ot[b], out_hbm.at[h, :, pl.ds(col, _BB)], sw[b])

        def w_wait(b):
            pltpu.make_async_copy(
                ot[b], out_hbm.at[0, :, pl.ds(col, _BB)], sw[b]).wait()

        def extract(b):
            pv = [plsc.load_gather(p64[b], [lanes[j]]) for j in range(8)]

            def fbody(f, carry):
                fv = jnp.full((16,), f, jnp.int32)
                for j in range(8):
                    val = plsc.load_gather(grows[b], [lanes[j], pv[j] + fv])
                    plsc.store_scatter(ot[b], [fv, lanes[j]], val)
                return carry

            lax.fori_loop(0, _D, fbody, 0)

        prep(0, 0)
        g_start(0)
        prep(1, 1)
        g_start(1)
        for h in (0, 1):
            b = h & 1
            g_wait(b)
            extract(b)
            w_start(h, b)
            prep(h + 2, b)
            g_start(b)

        def body(g, carry):
            for b in (0, 1):
                h = 2 * g + b
                g_wait(b)
                w_wait(b)
                extract(b)
                w_start(h, b)
                prep(h + 2, b)
                g_start(b)
            return carry

        lax.fori_loop(1, 99, body, 0)

        for h in (198, 199):
            b = h & 1
            g_wait(b)
            w_wait(b)
            extract(b)
            w_start(h, b)
        w_wait(0)
        w_wait(1)

    return k


def kernel(xs, table):
    table_r = table.reshape(500000, 128)
    out_t = _make()(xs.T, table_r)
    return out_t.transpose(2, 0, 1)


# speedup vs baseline: 1.1400x; 1.1400x over previous
import functools

import jax
import jax.numpy as jnp
from jax import lax
from jax.experimental import pallas as pl
from jax.experimental.pallas import tpu as pltpu
from jax.experimental.pallas import tpu_sc as plsc

_H = 200
_BATCH = 4096
_D = 64
_BB = 128


@functools.lru_cache(maxsize=None)
def _make():
    info = plsc.get_sparse_core_info()
    nw = info.num_cores * info.num_subcores
    assert nw * _BB == _BATCH

    mesh = plsc.VectorSubcoreMesh(core_axis_name="c", subcore_axis_name="s")

    @functools.partial(
        pl.kernel,
        mesh=mesh,
        out_type=jax.ShapeDtypeStruct((_H, 8, 32, 8, 128), jnp.float32),
        scratch_types=[
            pltpu.VMEM((_H, _BB), jnp.int32),
            pltpu.VMEM((_BB,), jnp.int32),
            pltpu.VMEM((_BB,), jnp.int32),
            pltpu.VMEM((_BB, _D), jnp.float32),
            pltpu.VMEM((_BB, _D), jnp.float32),
            pltpu.VMEM((_D, _BB), jnp.float32),
            pltpu.VMEM((_D, _BB), jnp.float32),
            pltpu.SemaphoreType.DMA,
            pltpu.SemaphoreType.DMA,
            pltpu.SemaphoreType.DMA,
            pltpu.SemaphoreType.DMA,
        ],
        compiler_params=pltpu.CompilerParams(
            use_tc_tiling_on_sc=False, needs_layout_passes=False),
    )
    def k(xst_hbm, table_hbm, out_hbm,
          idxt, i0, i1, g0, g1, o0, o1, sg0, sg1, sw0, sw1):
        wid = lax.axis_index("s") * info.num_cores + lax.axis_index("c")
        col = wid * _BB
        idx2 = (i0, i1)
        grows = (g0, g1)
        ot = (o0, o1)
        sg = (sg0, sg1)
        sw = (sw0, sw1)

        iota = lax.iota(jnp.int32, 16)
        lanes = [iota + 16 * j for j in range(8)]

        pltpu.sync_copy(xst_hbm.at[:, pl.ds(col, _BB)], idxt)

        def prep(h, b):
            hv = jnp.full((16,), h, jnp.int32)
            for j in range(8):
                v = plsc.load_gather(idxt, [hv, lanes[j]])
                idx2[b][pl.ds(16 * j, 16)] = v

        def g_start(b):
            pltpu.async_copy(table_hbm.at[idx2[b]], grows[b], sg[b])

        def g_wait(b):
            pltpu.make_async_copy(table_hbm.at[idx2[b]], grows[b], sg[b]).wait()

        def w_start(h, b):
            for fr in range(8):
                pltpu.async_copy(
                    ot[b].at[pl.ds(8 * fr, 8), :], out_hbm.at[h, fr, wid],
                    sw[b])

        def w_wait(b):
            for fr in range(8):
                pltpu.make_async_copy(
                    ot[b].at[pl.ds(8 * fr, 8), :], out_hbm.at[0, fr, wid],
                    sw[b]).wait()

        flanes = [iota + 16 * k for k in range(4)]

        def extract(b):
            def ibody(i, carry):
                iv = jnp.full((16,), i, jnp.int32)
                for kk in range(4):
                    val = grows[b][i, pl.ds(16 * kk, 16)]
                    plsc.store_scatter(ot[b], [flanes[kk], iv], val)
                return carry

            lax.fori_loop(0, _BB, ibody, 0)

        prep(0, 0)
        g_start(0)
        prep(1, 1)
        g_start(1)
        for h in (0, 1):
            b = h & 1
            g_wait(b)
            extract(b)
            w_start(h, b)
            prep(h + 2, b)
            g_start(b)

        def body(g, carry):
            for b in (0, 1):
                h = 2 * g + b
                g_wait(b)
                w_wait(b)
                extract(b)
                w_start(h, b)
                prep(h + 2, b)
                g_start(b)
            return carry

        lax.fori_loop(1, 99, body, 0)

        for h in (198, 199):
            b = h & 1
            g_wait(b)
            w_wait(b)
            extract(b)
            w_start(h, b)
        w_wait(0)
        w_wait(1)

    return k


def kernel(xs, table):
    out5 = _make()(xs.T, table)
    return out5.transpose(2, 4, 0, 1, 3).reshape(_BATCH, _H, _D)
